# Initial kernel scaffold; baseline (speedup 1.0000x reference)
import jax
import jax.numpy as jnp
from jax.experimental import pallas as pl


def kernel(q, k, v, Wc, bc, codebook, lengths, inv_lengths):
    raise NotImplementedError("write your pallas kernel here")


# speedup vs baseline: 5.5441x; 5.5441x over previous
import jax
import jax.numpy as jnp
from jax.experimental import pallas as pl

_EMBED = 1024
_HEADS = 16
_HD = _EMBED // _HEADS
_CS = 8
_K = 2 ** _CS
_SCALE = _HD ** -0.5


def _body(q_ref, k_ref, v_ref, wc_ref, bc_ref, cb_ref, o_ref):
    f32 = jnp.float32
    i32 = jnp.int32
    S = k_ref.shape[0]

    kb = k_ref[...]
    code = jax.lax.dot_general(kb, wc_ref[...], (((1,), (1,)), ((), ())),
                               preferred_element_type=f32)
    code = code + bc_ref[...]
    bits = (code >= 0.0).astype(i32)
    jj = jax.lax.broadcasted_iota(i32, (S, _CS), 1)
    pw = jax.lax.shift_left(jnp.ones((S, _CS), i32), (_CS - 1) - jj)
    idx = jnp.sum(bits * pw, axis=1, keepdims=True)

    onehot = (idx == jax.lax.broadcasted_iota(i32, (S, _K), 1)).astype(f32)
    codv = jax.lax.dot_general(onehot, v_ref[...], (((0,), (0,)), ((), ())),
                               preferred_element_type=f32)
    counts = jnp.sum(onehot, axis=0, keepdims=True)

    ii = jax.lax.broadcasted_iota(i32, (_K, 2 * _CS), 0)
    jj2 = jax.lax.broadcasted_iota(i32, (_K, 2 * _CS), 1)
    sh = jnp.where(jj2 < _CS, (_CS - 1) - jj2, (2 * _CS - 1) - jj2)
    bit = jax.lax.shift_right_logical(ii, sh) & 1
    sel = jnp.where(jj2 < _CS, bit, 1 - bit).astype(f32)
    codk = jax.lax.dot_general(sel, cb_ref[...], (((1,), (0,)), ((), ())),
                               preferred_element_type=f32)

    neg = jnp.where(counts > 0.0, 0.0, -1e30)
    qb = q_ref[...]
    for h in range(_HEADS):
        sl = slice(h * _HD, (h + 1) * _HD)
        logits = jax.lax.dot_general(qb[:, sl], codk[:, sl],
                                     (((1,), (1,)), ((), ())),
                                     preferred_element_type=f32) * _SCALE
        logits = logits + neg
        m = jnp.max(logits, axis=1, keepdims=True)
        e = jnp.exp(logits - m)
        num = jax.lax.dot_general(e, codv[:, sl], (((1,), (0,)), ((), ())),
                                  preferred_element_type=f32)
        den = jnp.sum(e * counts, axis=1, keepdims=True)
        o_ref[:, sl] = num / den


def kernel(q, k, v, Wc, bc, codebook, lengths, inv_lengths):
    L = q.shape[0]
    B = len(lengths)
    seg = L // B
    bc2 = bc.reshape(1, _CS)
    grid = (B,)
    blk = lambda b: (b, 0)
    fixed = lambda b: (0, 0)
    return pl.pallas_call(
        _body,
        grid=grid,
        in_specs=[
            pl.BlockSpec((seg, _EMBED), blk),
            pl.BlockSpec((seg, _EMBED), blk),
            pl.BlockSpec((seg, _EMBED), blk),
            pl.BlockSpec((_CS, _EMBED), fixed),
            pl.BlockSpec((1, _CS), fixed),
            pl.BlockSpec((2 * _CS, _EMBED), fixed),
        ],
        out_specs=pl.BlockSpec((seg, _EMBED), blk),
        out_shape=jax.ShapeDtypeStruct((L, _EMBED), jnp.float32),
    )(q, k, v, Wc, bc2, codebook)
